# baseline (speedup 1.0000x reference)
import functools

import jax
import jax.numpy as jnp
from jax import lax
from jax.experimental import pallas as pl
from jax.experimental.pallas import tpu as pltpu
from jax.experimental.pallas import tpu_sc as plsc

B = 4096
S = 200
HID = 64
TOT = B * S
NC, NS = 2, 16
NW = NC * NS
PER_W = TOT // NW
CH = 128
NCHUNK = PER_W // CH
NBUF = 8
LOOK = 4
NGRP = NCHUNK // NBUF
SCALE = 8.0
LANES = 16


def _emb_body(idx_hbm, table_hbm, out_hbm, idx_v, rows, gsem, osem):
    wid = lax.axis_index("s") * NC + lax.axis_index("c")
    base = wid * PER_W

    pltpu.sync_copy(idx_hbm.at[wid], idx_v)

    def start_gather(j, b):
        pltpu.async_copy(table_hbm.at[idx_v.at[j]], rows[b], gsem[b])

    def wait_gather(b):
        pltpu.make_async_copy(table_hbm.at[idx_v.at[0]], rows[b], gsem[b]).wait()

    def start_out(j, b):
        pltpu.async_copy(rows[b], out_hbm.at[pl.ds(base + j * CH, CH)], osem[b])

    def wait_out(b):
        pltpu.make_async_copy(rows[b], out_hbm.at[pl.ds(base, CH)], osem[b]).wait()

    for b in range(LOOK):
        start_gather(b, b)

    def group(g, _):
        for b in range(NBUF):
            j = g * NBUF + b
            bf = (b + LOOK) % NBUF
            if b < LOOK:
                @pl.when(g >= 1)
                def _():
                    wait_out(bf)
                start_gather(j + LOOK, bf)
            else:
                @pl.when(g < NGRP - 1)
                def _():
                    wait_out(bf)
                    start_gather(j + LOOK, bf)

            wait_gather(b)

            def scale_row(r, _):
                for c in range(HID // LANES):
                    sl = pl.ds(c * LANES, LANES)
                    rows[b][r, sl] = rows[b][r, sl] * SCALE
                return 0

            lax.fori_loop(0, CH, scale_row, 0, unroll=2)

            start_out(j, b)
        return 0

    lax.fori_loop(0, NGRP, group, 0)

    for b in range(NBUF):
        wait_out(b)


@jax.jit
def kernel(x, emb_weight):
    idx = x.reshape(NW, NCHUNK, CH).astype(jnp.int32)
    mesh = plsc.VectorSubcoreMesh(core_axis_name="c", subcore_axis_name="s")
    out = pl.kernel(
        _emb_body,
        out_type=jax.ShapeDtypeStruct((TOT, HID), jnp.float32),
        mesh=mesh,
        compiler_params=pltpu.CompilerParams(use_tc_tiling_on_sc=False),
        scratch_types=dict(
            idx_v=pltpu.VMEM((NCHUNK, CH), jnp.int32),
            rows=[pltpu.VMEM((CH, HID), jnp.float32) for _ in range(NBUF)],
            gsem=[pltpu.SemaphoreType.DMA for _ in range(NBUF)],
            osem=[pltpu.SemaphoreType.DMA for _ in range(NBUF)],
        ),
    )(idx, emb_weight)
    return out.reshape(B, S, HID)


# speedup vs baseline: 1.0897x; 1.0897x over previous
import functools

import jax
import jax.numpy as jnp
from jax import lax
from jax.experimental import pallas as pl
from jax.experimental.pallas import tpu as pltpu
from jax.experimental.pallas import tpu_sc as plsc

B = 4096
S = 200
HID = 64
NC, NS = 2, 16
NW = NC * NS
UNIT = 256
CH = 128
UNITS_PER_S = B // UNIT
NUNIT = S * UNITS_PER_S
UPW = NUNIT // NW
NT = UPW // 4
SCALE = 8.0
TB_C, TB_L = 3, 133


def _gather_desc(table_hbm, idxb, gbuf, gsem):
    return pltpu.make_async_copy(
        table_hbm.at[idxb.at[pl.ds(0, CH)]], gbuf, gsem)


def _out_descs(tbuf, o_hbm, osem):
    return [
        pltpu.make_async_copy(
            tbuf.at[ht, pl.ds(0, 2), :, pl.ds(0, CH)],
            o_hbm.at[0, ht, pl.ds(0, 2)],
            osem,
        )
        for ht in range(8)
    ]


def _emb_body(xT_hbm, table_hbm, o_hbm, idxb, gbuf, tbuf, isem, gsem, osem):
    wid = lax.axis_index("s") * NC + lax.axis_index("c")
    u0 = wid * UPW

    iota = lax.iota(jnp.int32, 16)
    r_vec = lax.bitwise_and(iota, 7)
    ht_half = lax.shift_right_logical(iota, 3)

    def fire_idx(u_local, buf):
        uu = u0 + u_local
        s = uu // UNITS_PER_S
        q = uu % UNITS_PER_S
        pltpu.async_copy(
            xT_hbm.at[s, pl.ds(q * UNIT, UNIT)], idxb[buf], isem[buf])

    def wait_idx(buf):
        pltpu.make_async_copy(
            xT_hbm.at[0, pl.ds(0, UNIT)], idxb[buf], isem[buf]).wait()

    def fire_gather(ibuf, g, b):
        pltpu.async_copy(
            table_hbm.at[idxb[ibuf].at[pl.ds(g * CH, CH)]], gbuf[b], gsem[b])

    def transpose_chunk(b, p, g):
        c_vec = jnp.full((16,), g, jnp.int32)

        def row(j2, _):
            l_vec = jnp.full((16,), 0, jnp.int32) + j2
            for m in range(4):
                ht_vec = ht_half + (2 * m)
                val = gbuf[b][j2, pl.ds(16 * m, 16)] * SCALE
                plsc.store_scatter(
                    tbuf[p], [ht_vec, c_vec, r_vec, l_vec], val)
            return 0

        lax.fori_loop(0, CH, row, 0, unroll=2)

    def fire_out(u_local, p):
        uu = u0 + u_local
        s = uu // UNITS_PER_S
        q = uu % UNITS_PER_S
        for ht in range(8):
            pltpu.async_copy(
                tbuf[p].at[ht, pl.ds(0, 2), :, pl.ds(0, CH)],
                o_hbm.at[s, ht, pl.ds(2 * q, 2)],
                osem[p],
            )

    def wait_out(p):
        for d in _out_descs(tbuf[p], o_hbm, osem[p]):
            d.wait()

    for uu in range(4):
        fire_idx(uu, uu)
    for uu in range(3):
        wait_idx(uu)
        for g in range(2):
            fire_gather(uu, g, 2 * uu + g)

    def block(t, _):
        for k in range(4):
            u_rel = 4 * t + k
            p = k % 2
            for g in range(2):
                pos = 2 * k + g
                if g == 0:
                    @pl.when(t < NT - 1)
                    def _():
                        fire_idx(u_rel + 4, k)
                if pos < 2:
                    if g == 0:
                        wait_idx(3)
                    fire_gather(3, g, pos + 6)
                else:
                    @pl.when(t < NT - 1)
                    def _():
                        if g == 0:
                            wait_idx(k - 1)
                        fire_gather(k - 1, g, pos - 2)

                if g == 0:
                    if k < 2:
                        @pl.when(t >= 1)
                        def _():
                            wait_out(p)
                    else:
                        wait_out(p)

                _gather_desc(table_hbm, idxb[0], gbuf[pos], gsem[pos]).wait()
                transpose_chunk(pos, p, g)

            fire_out(u_rel, p)
        return 0

    lax.fori_loop(0, NT, block, 0)

    wait_out(0)
    wait_out(1)


@jax.jit
def kernel(x, emb_weight):
    xT = jnp.transpose(x).astype(jnp.int32)
    mesh = plsc.VectorSubcoreMesh(core_axis_name="c", subcore_axis_name="s")
    O = pl.kernel(
        _emb_body,
        out_type=jax.ShapeDtypeStruct((S, 8, B // CH, 8, CH), jnp.float32),
        mesh=mesh,
        compiler_params=pltpu.CompilerParams(
            use_tc_tiling_on_sc=False, needs_layout_passes=False),
        scratch_types=dict(
            idxb=[pltpu.VMEM((UNIT,), jnp.int32) for _ in range(4)],
            gbuf=[pltpu.VMEM((CH, HID), jnp.float32) for _ in range(8)],
            tbuf=[pltpu.VMEM((8, TB_C, 8, TB_L), jnp.float32) for _ in range(2)],
            isem=[pltpu.SemaphoreType.DMA for _ in range(4)],
            gsem=[pltpu.SemaphoreType.DMA for _ in range(8)],
            osem=[pltpu.SemaphoreType.DMA for _ in range(2)],
        ),
    )(xT, emb_weight)
    return O.transpose(2, 4, 0, 1, 3).reshape(B, S, HID)
